# Initial kernel scaffold; baseline (speedup 1.0000x reference)
import jax
import jax.numpy as jnp
from jax.experimental import pallas as pl


def kernel(x, edge_index, W1, b1, W2, b2):
    raise NotImplementedError("write your pallas kernel here")


# speedup vs baseline: 14.9250x; 14.9250x over previous
import functools

import jax
import jax.numpy as jnp
from jax import lax
from jax.experimental import pallas as pl
from jax.experimental.pallas import tpu as pltpu
from jax.experimental.pallas import tpu_sc as plsc

_NC = 2
_NS = 16
_LANE = 128
_RB = 1024


def _sc_degree(dst2d, npad, nblk_w):
    rows_t = npad // _NS
    mesh = plsc.VectorSubcoreMesh(core_axis_name="c", subcore_axis_name="s")

    @functools.partial(
        pl.kernel,
        out_type=jax.ShapeDtypeStruct((_NC, npad), jnp.float32),
        mesh=mesh,
        scratch_types=[
            pltpu.VMEM((nblk_w, _LANE), jnp.int32),
            pltpu.VMEM((_LANE,), jnp.float32),
            pltpu.VMEM((rows_t,), jnp.float32),
            pltpu.VMEM_SHARED((npad,), jnp.float32),
        ],
    )
    def k(dst_hbm, out_hbm, dst_v, ones_v, init_v, acc):
        cid = lax.axis_index("c")
        sid = lax.axis_index("s")
        wid = cid * _NS + sid

        @pl.loop(0, _LANE, step=16)
        def _(i):
            ones_v[pl.ds(i, 16)] = jnp.ones((16,), jnp.float32)

        val = jnp.where(cid == 0, jnp.float32(1.0), jnp.float32(0.0))

        @pl.loop(0, rows_t, step=16)
        def _(i):
            init_v[pl.ds(i, 16)] = jnp.zeros((16,), jnp.float32) + val

        pltpu.sync_copy(init_v, acc.at[pl.ds(sid * rows_t, rows_t)])
        plsc.subcore_barrier()

        pltpu.sync_copy(dst_hbm.at[wid], dst_v)

        @pl.loop(0, nblk_w)
        def _(j):
            pltpu.sync_copy(ones_v, acc.at[dst_v.at[j]], add=True)

        plsc.subcore_barrier()
        pltpu.sync_copy(acc.at[pl.ds(sid * rows_t, rows_t)],
                        out_hbm.at[cid, pl.ds(sid * rows_t, rows_t)])

    return k(dst2d)


def _sc_segsum(g, src2d, dst2d, nblk_w):
    npad, d = g.shape
    rows_t = npad // _NS
    nchunks = rows_t // _LANE
    mesh = plsc.VectorSubcoreMesh(core_axis_name="c", subcore_axis_name="s")

    @functools.partial(
        pl.kernel,
        out_type=jax.ShapeDtypeStruct((_NC, npad, d), jnp.float32),
        mesh=mesh,
        scratch_types=[
            pltpu.VMEM((nblk_w, _LANE), jnp.int32),
            pltpu.VMEM((nblk_w, _LANE), jnp.int32),
            pltpu.VMEM((_LANE, d), jnp.float32),
            pltpu.VMEM_SHARED((npad, d), jnp.float32),
        ],
        compiler_params=pltpu.CompilerParams(use_tc_tiling_on_sc=False),
    )
    def k(g_hbm, src_hbm, dst_hbm, out_hbm, src_v, dst_v, rows_v, acc):
        cid = lax.axis_index("c")
        sid = lax.axis_index("s")
        wid = cid * _NS + sid

        @pl.loop(0, _LANE)
        def _(i):
            @pl.loop(0, d, step=16)
            def _(j):
                rows_v[i, pl.ds(j, 16)] = jnp.zeros((16,), jnp.float32)

        base = sid * rows_t

        @pl.loop(0, nchunks)
        def _(t):
            pltpu.sync_copy(rows_v, acc.at[pl.ds(base + t * _LANE, _LANE)])

        plsc.subcore_barrier()

        pltpu.sync_copy(src_hbm.at[wid], src_v)
        pltpu.sync_copy(dst_hbm.at[wid], dst_v)

        @pl.loop(0, nblk_w)
        def _(j):
            pltpu.sync_copy(g_hbm.at[src_v.at[j]], rows_v)
            pltpu.sync_copy(rows_v, acc.at[dst_v.at[j]], add=True)

        plsc.subcore_barrier()

        @pl.loop(0, nchunks)
        def _(t):
            pltpu.sync_copy(acc.at[pl.ds(base + t * _LANE, _LANE)],
                            out_hbm.at[cid, pl.ds(base + t * _LANE, _LANE)])

    return k(g, src2d, dst2d)


def _dinv_of(deg_ref):
    return lax.rsqrt(jnp.maximum(deg_ref[0] + deg_ref[1], 1.0))


def _l1_body(x_ref, w_ref, deg_ref, g_ref):
    dinv = _dinv_of(deg_ref)
    y = jnp.dot(x_ref[...], w_ref[...], preferred_element_type=jnp.float32)
    g_ref[...] = y * dinv


def _l2_body(s_ref, g1_ref, deg_ref, b1_ref, w2_ref, g2_ref):
    dinv = _dinv_of(deg_ref)
    agg = dinv * (s_ref[0] + s_ref[1] + g1_ref[...]) + b1_ref[...]
    h = jnp.tanh(agg)
    g2_ref[...] = jnp.dot(h, w2_ref[...], preferred_element_type=jnp.float32) * dinv


def _l3_body(s_ref, g2_ref, deg_ref, b2_ref, o_ref):
    dinv = _dinv_of(deg_ref)
    o_ref[...] = dinv * (s_ref[0] + s_ref[1] + g2_ref[...]) + b2_ref[...]


def _tc_layer1(xp, W1, deg3):
    npad, di = xp.shape
    dh = W1.shape[1]
    return pl.pallas_call(
        _l1_body,
        grid=(npad // _RB,),
        in_specs=[
            pl.BlockSpec((_RB, di), lambda i: (i, 0)),
            pl.BlockSpec((di, dh), lambda i: (0, 0)),
            pl.BlockSpec((_NC, _RB, 1), lambda i: (0, i, 0)),
        ],
        out_specs=pl.BlockSpec((_RB, dh), lambda i: (i, 0)),
        out_shape=jax.ShapeDtypeStruct((npad, dh), jnp.float32),
    )(xp, W1, deg3)


def _tc_layer2(s1, g1, deg3, b1, W2):
    npad, dh = g1.shape
    do = W2.shape[1]
    return pl.pallas_call(
        _l2_body,
        grid=(npad // _RB,),
        in_specs=[
            pl.BlockSpec((_NC, _RB, dh), lambda i: (0, i, 0)),
            pl.BlockSpec((_RB, dh), lambda i: (i, 0)),
            pl.BlockSpec((_NC, _RB, 1), lambda i: (0, i, 0)),
            pl.BlockSpec((1, dh), lambda i: (0, 0)),
            pl.BlockSpec((dh, do), lambda i: (0, 0)),
        ],
        out_specs=pl.BlockSpec((_RB, do), lambda i: (i, 0)),
        out_shape=jax.ShapeDtypeStruct((npad, do), jnp.float32),
    )(s1, g1, deg3, b1, W2)


def _tc_layer3(s2, g2, deg3, b2):
    npad, do = g2.shape
    return pl.pallas_call(
        _l3_body,
        grid=(npad // _RB,),
        in_specs=[
            pl.BlockSpec((_NC, _RB, do), lambda i: (0, i, 0)),
            pl.BlockSpec((_RB, do), lambda i: (i, 0)),
            pl.BlockSpec((_NC, _RB, 1), lambda i: (0, i, 0)),
            pl.BlockSpec((1, do), lambda i: (0, 0)),
        ],
        out_specs=pl.BlockSpec((_RB, do), lambda i: (i, 0)),
        out_shape=jax.ShapeDtypeStruct((npad, do), jnp.float32),
    )(s2, g2, deg3, b2)


def kernel(x, edge_index, W1, b1, W2, b2):
    n, di = x.shape
    dh = W1.shape[1]
    do = W2.shape[1]
    e = edge_index.shape[1]

    blk = _NS * _LANE
    npad = ((n + blk - 1) // blk) * blk
    nblk_w = (e + _NC * _NS * _LANE - 1) // (_NC * _NS * _LANE)
    epad = nblk_w * _NC * _NS * _LANE

    src = edge_index[0]
    dst = edge_index[1]
    srcp = jnp.concatenate(
        [src, jnp.zeros((epad - e,), src.dtype)]).reshape(_NC * _NS, nblk_w, _LANE)
    dstp = jnp.concatenate(
        [dst, jnp.full((epad - e,), n, dst.dtype)]).reshape(_NC * _NS, nblk_w, _LANE)
    xp = jnp.pad(x, ((0, npad - n), (0, 0)))

    deg2 = _sc_degree(dstp, npad, nblk_w)
    deg3 = deg2[:, :, None]

    g1 = _tc_layer1(xp, W1, deg3)
    s1 = _sc_segsum(g1, srcp, dstp, nblk_w)
    g2 = _tc_layer2(s1, g1, deg3, b1.reshape(1, dh), W2)
    s2 = _sc_segsum(g2, srcp, dstp, nblk_w)
    out = _tc_layer3(s2, g2, deg3, b2.reshape(1, do))
    return out[:n]
